# baseline (speedup 1.0000x reference)
import functools

import jax
import jax.numpy as jnp
from jax import lax
from jax.experimental import pallas as pl
from jax.experimental.pallas import tpu as pltpu
from jax.experimental.pallas import tpu_sc as plsc

F = 9
D = 128
L = 16
NC = 2
NS = 16
NW = NC * NS
C = 40
G = 120
NSTR = C * F // G


def _body(n_pad, table, idxf, w_hbm, b_hbm, out_hbm,
          w_v, b_v, idx_v0, idx_v1, rows_v0, rows_v1, out_v0, out_v1,
          isem0, isem1, gsem0, gsem1, osem0, osem1):
    b_per_w = n_pad // NW
    n_chunks = b_per_w // C
    idx_v = (idx_v0, idx_v1)
    rows_v = (rows_v0, rows_v1)
    out_v = (out_v0, out_v1)
    isem = (isem0, isem1)
    gsem = (gsem0, gsem1)
    osem = (osem0, osem1)
    wid = lax.axis_index("s") * NC + lax.axis_index("c")
    vbase = wid * b_per_w

    pltpu.sync_copy(w_hbm, w_v)
    pltpu.sync_copy(b_hbm, b_v)

    def fire_gathers(u, slot):
        i0 = pl.multiple_of((vbase + u * C) * F, C * F)
        pltpu.async_copy(idxf.at[pl.ds(i0, C * F)], idx_v[slot], isem[slot])
        pltpu.make_async_copy(idxf.at[pl.ds(0, C * F)], idx_v[slot],
                              isem[slot]).wait()
        for j in range(NSTR):
            pltpu.async_copy(
                table.at[idx_v[slot].at[pl.ds(j * G, G)]],
                rows_v[slot].at[pl.ds(j * G, G)],
                gsem[slot],
            )

    fire_gathers(0, 0)

    def pair_body(t, carry):
        for b in range(2):
            u = 2 * t + b
            nb = 1 - b
            @pl.when(u + 1 < n_chunks)
            def _():
                fire_gathers(u + 1, nb)
            pltpu.make_async_copy(table.at[pl.ds(0, C * F)], rows_v[b],
                                  gsem[b]).wait()
            @pl.when(u >= 2)
            def _():
                pltpu.make_async_copy(out_v[b], out_hbm.at[pl.ds(0, C)],
                                      osem[b]).wait()

            rows = rows_v[b]
            outc = out_v[b]

            def c_body(c, carry2):
                r0 = c * F
                for g in range(D // L):
                    sl = pl.ds(g * L, L)
                    acc = b_v[sl]
                    for f in range(F):
                        acc = acc + rows[r0 + f, sl] * w_v[f, sl]
                    outc[c, sl] = acc
                return carry2

            lax.fori_loop(0, C, c_body, 0)
            v0 = pl.multiple_of(vbase + u * C, C)
            pltpu.async_copy(out_v[b], out_hbm.at[pl.ds(v0, C)], osem[b])
        return carry

    lax.fori_loop(0, n_chunks // 2, pair_body, 0)
    for b in range(2):
        pltpu.make_async_copy(out_v[b], out_hbm.at[pl.ds(0, C)],
                              osem[b]).wait()


def kernel(lattice_values, neighbor_idx, weight, bias):
    n = lattice_values.shape[0]
    n_pad = -(-n // (NW * 2 * C)) * (NW * 2 * C)
    idxf = neighbor_idx.astype(jnp.int32).reshape(-1)
    if n_pad != n:
        idxf = jnp.pad(idxf, (0, (n_pad - n) * F))

    mesh = plsc.VectorSubcoreMesh(core_axis_name="c", subcore_axis_name="s")
    run = pl.kernel(
        functools.partial(_body, n_pad),
        out_type=jax.ShapeDtypeStruct((n_pad, D), jnp.float32),
        mesh=mesh,
        scratch_types=[
            pltpu.VMEM((F, D), jnp.float32),
            pltpu.VMEM((D,), jnp.float32),
            pltpu.VMEM((C * F,), jnp.int32),
            pltpu.VMEM((C * F,), jnp.int32),
            pltpu.VMEM((C * F, D), jnp.float32),
            pltpu.VMEM((C * F, D), jnp.float32),
            pltpu.VMEM((C, D), jnp.float32),
            pltpu.VMEM((C, D), jnp.float32),
            pltpu.SemaphoreType.DMA,
            pltpu.SemaphoreType.DMA,
            pltpu.SemaphoreType.DMA,
            pltpu.SemaphoreType.DMA,
            pltpu.SemaphoreType.DMA,
            pltpu.SemaphoreType.DMA,
        ],
    )
    out = run(lattice_values, idxf, weight, bias)
    return out[:n]


# speedup vs baseline: 1.0014x; 1.0014x over previous
import functools

import jax
import jax.numpy as jnp
from jax import lax
from jax.experimental import pallas as pl
from jax.experimental.pallas import tpu as pltpu
from jax.experimental.pallas import tpu_sc as plsc

F = 9
D = 128
L = 16
NC = 2
NS = 16
NW = NC * NS
C = 32
G = 96
NSTR = C * F // G


def _body(n_pad, table, idxf, w_hbm, b_hbm, out_hbm,
          w_v, b_v, idx_v0, idx_v1, rows_v0, rows_v1, out_v0, out_v1,
          isem0, isem1, gsem0, gsem1, osem0, osem1):
    b_per_w = n_pad // NW
    n_chunks = b_per_w // C
    idx_v = (idx_v0, idx_v1)
    rows_v = (rows_v0, rows_v1)
    out_v = (out_v0, out_v1)
    isem = (isem0, isem1)
    gsem = (gsem0, gsem1)
    osem = (osem0, osem1)
    wid = lax.axis_index("s") * NC + lax.axis_index("c")
    vbase = wid * b_per_w

    pltpu.sync_copy(w_hbm, w_v)
    pltpu.sync_copy(b_hbm, b_v)

    def fire_gathers(u, slot):
        i0 = pl.multiple_of((vbase + u * C) * F, C * F)
        pltpu.async_copy(idxf.at[pl.ds(i0, C * F)], idx_v[slot], isem[slot])
        pltpu.make_async_copy(idxf.at[pl.ds(0, C * F)], idx_v[slot],
                              isem[slot]).wait()
        for j in range(NSTR):
            pltpu.async_copy(
                table.at[idx_v[slot].at[pl.ds(j * G, G)]],
                rows_v[slot].at[pl.ds(j * G, G)],
                gsem[slot],
            )

    fire_gathers(0, 0)

    def pair_body(t, carry):
        for b in range(2):
            u = 2 * t + b
            nb = 1 - b
            @pl.when(u + 1 < n_chunks)
            def _():
                fire_gathers(u + 1, nb)
            pltpu.make_async_copy(table.at[pl.ds(0, C * F)], rows_v[b],
                                  gsem[b]).wait()
            @pl.when(u >= 2)
            def _():
                pltpu.make_async_copy(out_v[b], out_hbm.at[pl.ds(0, C)],
                                      osem[b]).wait()

            rows = rows_v[b]
            outc = out_v[b]

            def c_body(c, carry2):
                r0 = c * F
                for g in range(D // L):
                    sl = pl.ds(g * L, L)
                    acc = b_v[sl]
                    for f in range(F):
                        acc = acc + rows[r0 + f, sl] * w_v[f, sl]
                    outc[c, sl] = acc
                return carry2

            lax.fori_loop(0, C, c_body, 0)
            v0 = pl.multiple_of(vbase + u * C, C)
            pltpu.async_copy(out_v[b], out_hbm.at[pl.ds(v0, C)], osem[b])
        return carry

    lax.fori_loop(0, n_chunks // 2, pair_body, 0)
    for b in range(2):
        pltpu.make_async_copy(out_v[b], out_hbm.at[pl.ds(0, C)],
                              osem[b]).wait()


def kernel(lattice_values, neighbor_idx, weight, bias):
    n = lattice_values.shape[0]
    n_pad = -(-n // (NW * 2 * C)) * (NW * 2 * C)
    idxf = neighbor_idx.astype(jnp.int32).reshape(-1)
    if n_pad != n:
        idxf = jnp.pad(idxf, (0, (n_pad - n) * F))

    mesh = plsc.VectorSubcoreMesh(core_axis_name="c", subcore_axis_name="s")
    run = pl.kernel(
        functools.partial(_body, n_pad),
        out_type=jax.ShapeDtypeStruct((n_pad, D), jnp.float32),
        mesh=mesh,
        scratch_types=[
            pltpu.VMEM((F, D), jnp.float32),
            pltpu.VMEM((D,), jnp.float32),
            pltpu.VMEM((C * F,), jnp.int32),
            pltpu.VMEM((C * F,), jnp.int32),
            pltpu.VMEM((C * F, D), jnp.float32),
            pltpu.VMEM((C * F, D), jnp.float32),
            pltpu.VMEM((C, D), jnp.float32),
            pltpu.VMEM((C, D), jnp.float32),
            pltpu.SemaphoreType.DMA,
            pltpu.SemaphoreType.DMA,
            pltpu.SemaphoreType.DMA,
            pltpu.SemaphoreType.DMA,
            pltpu.SemaphoreType.DMA,
            pltpu.SemaphoreType.DMA,
        ],
    )
    out = run(lattice_values, idxf, weight, bias)
    return out[:n]
